# baseline (speedup 1.0000x reference)
import functools

import jax
import jax.numpy as jnp
from jax import lax
from jax.experimental import pallas as pl
from jax.experimental.pallas import tpu as pltpu
from jax.experimental.pallas import tpu_sc as plsc

NC = 2
NS = 16
NW = NC * NS
LN = 16
BM = 512
SUP = 16
KPT = 4


def _make_deg_kernel(NP, RA):
    mesh = plsc.VectorSubcoreMesh(core_axis_name="c", subcore_axis_name="s",
                                  num_cores=NC, num_subcores=NS)

    @functools.partial(
        pl.kernel, mesh=mesh,
        out_type=jax.ShapeDtypeStruct((NW, 2, NP), jnp.float32),
        scratch_types=[
            pltpu.VMEM((RA, 128), jnp.int32),
            pltpu.VMEM((RA, 128), jnp.int32),
            pltpu.VMEM((NP,), jnp.float32),
            pltpu.VMEM((NP,), jnp.float32),
        ],
        compiler_params=pltpu.CompilerParams(needs_layout_passes=False),
    )
    def deg_kernel(src_hbm, dst_hbm, out_hbm, sidx, didx, dego, degi):
        cid = lax.axis_index("c")
        sid = lax.axis_index("s")
        wid = sid * NC + cid
        pltpu.sync_copy(src_hbm.at[pl.ds(wid * RA, RA)], sidx)
        pltpu.sync_copy(dst_hbm.at[pl.ds(wid * RA, RA)], didx)
        zeros16 = jnp.zeros((LN,), jnp.float32)

        @plsc.parallel_loop(0, NP // LN, step=1, unroll=4)
        def zbody(i):
            dego[pl.ds(i * LN, LN)] = zeros16
            degi[pl.ds(i * LN, LN)] = zeros16

        ones16 = jnp.full((LN,), 1.0, jnp.float32)

        @plsc.parallel_loop(0, RA, step=1, unroll=2)
        def ebody(j):
            for g in range(128 // LN):
                si = sidx[j, pl.ds(g * LN, LN)]
                plsc.addupdate_scatter(dego, [si], ones16)
                di = didx[j, pl.ds(g * LN, LN)]
                plsc.addupdate_scatter(degi, [di], ones16)

        pltpu.sync_copy(dego, out_hbm.at[wid, 0])
        pltpu.sync_copy(degi, out_hbm.at[wid, 1])

    return deg_kernel


def _make_agg_kernel(NP, NR):
    NSS = NR // SUP
    mesh = plsc.VectorSubcoreMesh(core_axis_name="c", subcore_axis_name="s",
                                  num_cores=NC, num_subcores=NS)

    @functools.partial(
        pl.kernel, mesh=mesh,
        out_type=jax.ShapeDtypeStruct((NW, KPT, NP), jnp.float32),
        scratch_types=[pltpu.VMEM((NP,), jnp.float32)] * (2 * KPT) + [
            pltpu.VMEM((2, SUP, 128), jnp.int32),
            pltpu.VMEM((2, SUP, 128), jnp.int32),
            pltpu.SemaphoreType.DMA,
            pltpu.SemaphoreType.DMA,
        ],
        compiler_params=pltpu.CompilerParams(needs_layout_passes=False),
    )
    def agg_kernel(hs_hbm, src_hbm, dst_hbm, out_hbm, *rest):
        hcol = rest[:KPT]
        acol = rest[KPT:2 * KPT]
        sbuf, dbuf, ssem, dsem = rest[2 * KPT:]
        cid = lax.axis_index("c")
        sid = lax.axis_index("s")
        wid = sid * NC + cid

        for k in range(KPT):
            pltpu.sync_copy(hs_hbm.at[wid, k], hcol[k])

        zeros16 = jnp.zeros((LN,), jnp.float32)

        @plsc.parallel_loop(0, NP // LN, step=1, unroll=4)
        def zbody(i):
            for k in range(KPT):
                acol[k][pl.ds(i * LN, LN)] = zeros16

        pltpu.sync_copy(src_hbm.at[pl.ds(0, SUP)], sbuf.at[0])
        pltpu.sync_copy(dst_hbm.at[pl.ds(0, SUP)], dbuf.at[0])

        def body(g, _):
            gmod = g % 2

            @pl.when(g > 0)
            def _():
                pltpu.make_async_copy(src_hbm.at[pl.ds(0, SUP)],
                                      sbuf.at[0], ssem).wait()
                pltpu.make_async_copy(dst_hbm.at[pl.ds(0, SUP)],
                                      dbuf.at[0], dsem).wait()

            @pl.when(g < NSS - 1)
            def _():
                off = pl.multiple_of((g + 1) * SUP, SUP)
                nxt = (g + 1) % 2
                pltpu.async_copy(src_hbm.at[pl.ds(off, SUP)],
                                 sbuf.at[nxt], ssem)
                pltpu.async_copy(dst_hbm.at[pl.ds(off, SUP)],
                                 dbuf.at[nxt], dsem)

            @plsc.parallel_loop(0, SUP * (128 // LN), step=1, unroll=8)
            def gbody(t):
                r = lax.shift_right_logical(t, 3)
                s = pl.ds(lax.shift_left(lax.bitwise_and(t, 7), 4), LN)
                sv = sbuf[gmod, r, s]
                dv = dbuf[gmod, r, s]
                for k in range(KPT):
                    vals = plsc.load_gather(hcol[k], [sv])
                    plsc.addupdate_scatter(acol[k], [dv], vals)
            return 0
        lax.fori_loop(0, NSS, body, 0)

        for k in range(KPT):
            pltpu.sync_copy(acol[k], out_hbm.at[wid, k])

    return agg_kernel


def _norms_from_parts(dp):
    deg = jnp.sum(dp, axis=0)
    nrm = lax.rsqrt(jnp.maximum(deg, 1.0))
    return nrm[0:1, :], nrm[1:2, :]


def _mm_scale_body(x_ref, w_ref, dp_ref, o_ref):
    ns, _ = _norms_from_parts(dp_ref[...])
    y = lax.dot_general(w_ref[...], x_ref[...], (((0,), (1,)), ((), ())),
                        preferred_element_type=jnp.float32)
    o_ref[...] = y * ns


def _post_mm_body(a_ref, dp_ref, b_ref, w_ref, o_ref):
    ns, nd = _norms_from_parts(dp_ref[...])
    x = jnp.maximum(a_ref[...] * nd + b_ref[...], 0.0)
    y = lax.dot_general(w_ref[...], x, (((0,), (0,)), ((), ())),
                        preferred_element_type=jnp.float32)
    o_ref[...] = y * ns


def _make_readout_body(NN, NB, H, C):
    def readout_body(a_ref, dp_ref, b_ref, wr_ref, o_ref, acc_ref):
        i = pl.program_id(0)
        _, nd = _norms_from_parts(dp_ref[...])
        x = jnp.maximum(a_ref[...] * nd + b_ref[...], 0.0)
        colid = i * BM + lax.broadcasted_iota(jnp.int32, (H, BM), 1)
        x = jnp.where(colid < NN, x, 0.0)
        s = jnp.sum(x, axis=1, keepdims=True)

        @pl.when(i == 0)
        def _():
            acc_ref[...] = s

        @pl.when(i > 0)
        def _():
            acc_ref[...] = acc_ref[...] + s

        @pl.when(i == NB - 1)
        def _():
            o_ref[...] = lax.dot_general(
                acc_ref[...] / NN, wr_ref[...], (((0,), (0,)), ((), ())),
                preferred_element_type=jnp.float32)
    return readout_body


def kernel(feat, edge_index, W0, b0, W1, b1, Wr):
    NN, D = feat.shape
    E = edge_index.shape[1]
    H = W0.shape[1]
    C = Wr.shape[1]

    NP = -(-(NN + 1) // BM) * BM
    EP = -(-E // (NW * 128 * 16)) * (NW * 128 * 16)
    NR = EP // 128
    RA = NR // NW
    NB = NP // BM

    src = edge_index[0]
    dst = edge_index[1]
    padi = jnp.full((EP - E,), NN, jnp.int32)
    src2d = jnp.concatenate([src, padi]).reshape(NR, 128)
    dst2d = jnp.concatenate([dst, padi]).reshape(NR, 128)
    feat_p = jnp.pad(feat, ((0, NP - NN), (0, 0)))

    dparts = _make_deg_kernel(NP, RA)(src2d, dst2d)

    colT_spec = pl.BlockSpec((H, BM), lambda i: (0, i))
    dp_spec = pl.BlockSpec((NW, 2, BM), lambda i: (0, 0, i))
    w_spec = pl.BlockSpec((D, H), lambda i: (0, 0))
    bT_spec = pl.BlockSpec((H, 1), lambda i: (0, 0))

    hs1 = pl.pallas_call(
        _mm_scale_body,
        grid=(NB,),
        in_specs=[pl.BlockSpec((BM, D), lambda i: (i, 0)), w_spec, dp_spec],
        out_specs=colT_spec,
        out_shape=jax.ShapeDtypeStruct((H, NP), jnp.float32),
    )(feat_p, W0, dparts)

    agg = _make_agg_kernel(NP, NR)
    p1 = agg(hs1.reshape(NW, KPT, NP), src2d, dst2d)

    hs2 = pl.pallas_call(
        _post_mm_body,
        grid=(NB,),
        in_specs=[colT_spec, dp_spec, bT_spec, w_spec],
        out_specs=colT_spec,
        out_shape=jax.ShapeDtypeStruct((H, NP), jnp.float32),
    )(p1.reshape(H, NP), dparts, b0.reshape(H, 1), W1)

    p2 = agg(hs2.reshape(NW, KPT, NP), src2d, dst2d)

    out = pl.pallas_call(
        _make_readout_body(NN, NB, H, C),
        grid=(NB,),
        in_specs=[colT_spec, dp_spec, bT_spec,
                  pl.BlockSpec((H, C), lambda i: (0, 0))],
        out_specs=pl.BlockSpec((1, C), lambda i: (0, 0)),
        out_shape=jax.ShapeDtypeStruct((1, C), jnp.float32),
        scratch_shapes=[pltpu.VMEM((H, 1), jnp.float32)],
    )(p2.reshape(H, NP), dparts, b1.reshape(H, 1), Wr)

    return out


# speedup vs baseline: 1.0359x; 1.0359x over previous
import functools

import jax
import jax.numpy as jnp
from jax import lax
from jax.experimental import pallas as pl
from jax.experimental.pallas import tpu as pltpu
from jax.experimental.pallas import tpu_sc as plsc

NC = 2
NS = 16
NW = NC * NS
LN = 16
BM = 512
SUP = 32
KPT = 4


def _make_deg_kernel(NP, RA):
    mesh = plsc.VectorSubcoreMesh(core_axis_name="c", subcore_axis_name="s",
                                  num_cores=NC, num_subcores=NS)

    @functools.partial(
        pl.kernel, mesh=mesh,
        out_type=jax.ShapeDtypeStruct((NW, 2, NP), jnp.float32),
        scratch_types=[
            pltpu.VMEM((RA, 128), jnp.int32),
            pltpu.VMEM((RA, 128), jnp.int32),
            pltpu.VMEM((NP,), jnp.float32),
            pltpu.VMEM((NP,), jnp.float32),
        ],
        compiler_params=pltpu.CompilerParams(needs_layout_passes=False),
    )
    def deg_kernel(src_hbm, dst_hbm, out_hbm, sidx, didx, dego, degi):
        cid = lax.axis_index("c")
        sid = lax.axis_index("s")
        wid = sid * NC + cid
        pltpu.sync_copy(src_hbm.at[pl.ds(wid * RA, RA)], sidx)
        pltpu.sync_copy(dst_hbm.at[pl.ds(wid * RA, RA)], didx)
        zeros16 = jnp.zeros((LN,), jnp.float32)

        @plsc.parallel_loop(0, NP // LN, step=1, unroll=4)
        def zbody(i):
            dego[pl.ds(i * LN, LN)] = zeros16
            degi[pl.ds(i * LN, LN)] = zeros16

        ones16 = jnp.full((LN,), 1.0, jnp.float32)

        @plsc.parallel_loop(0, RA, step=1, unroll=2)
        def ebody(j):
            for g in range(128 // LN):
                si = sidx[j, pl.ds(g * LN, LN)]
                plsc.addupdate_scatter(dego, [si], ones16)
                di = didx[j, pl.ds(g * LN, LN)]
                plsc.addupdate_scatter(degi, [di], ones16)

        pltpu.sync_copy(dego, out_hbm.at[wid, 0])
        pltpu.sync_copy(degi, out_hbm.at[wid, 1])

    return deg_kernel


def _make_agg_kernel(NP, NR):
    NSS = NR // SUP
    mesh = plsc.VectorSubcoreMesh(core_axis_name="c", subcore_axis_name="s",
                                  num_cores=NC, num_subcores=NS)

    @functools.partial(
        pl.kernel, mesh=mesh,
        out_type=jax.ShapeDtypeStruct((NW, KPT, NP), jnp.float32),
        scratch_types=[pltpu.VMEM((NP,), jnp.float32)] * (2 * KPT) + [
            pltpu.VMEM((2, SUP, 128), jnp.int32),
            pltpu.VMEM((2, SUP, 128), jnp.int32),
            pltpu.SemaphoreType.DMA,
            pltpu.SemaphoreType.DMA,
        ],
        compiler_params=pltpu.CompilerParams(needs_layout_passes=False),
    )
    def agg_kernel(hs_hbm, src_hbm, dst_hbm, out_hbm, *rest):
        hcol = rest[:KPT]
        acol = rest[KPT:2 * KPT]
        sbuf, dbuf, ssem, dsem = rest[2 * KPT:]
        cid = lax.axis_index("c")
        sid = lax.axis_index("s")
        wid = sid * NC + cid

        for k in range(KPT):
            pltpu.sync_copy(hs_hbm.at[wid, k], hcol[k])

        zeros16 = jnp.zeros((LN,), jnp.float32)

        @plsc.parallel_loop(0, NP // LN, step=1, unroll=4)
        def zbody(i):
            for k in range(KPT):
                acol[k][pl.ds(i * LN, LN)] = zeros16

        pltpu.sync_copy(src_hbm.at[pl.ds(0, SUP)], sbuf.at[0])
        pltpu.sync_copy(dst_hbm.at[pl.ds(0, SUP)], dbuf.at[0])

        def body(g, _):
            gmod = g % 2

            @pl.when(g > 0)
            def _():
                pltpu.make_async_copy(src_hbm.at[pl.ds(0, SUP)],
                                      sbuf.at[0], ssem).wait()
                pltpu.make_async_copy(dst_hbm.at[pl.ds(0, SUP)],
                                      dbuf.at[0], dsem).wait()

            @pl.when(g < NSS - 1)
            def _():
                off = pl.multiple_of((g + 1) * SUP, SUP)
                nxt = (g + 1) % 2
                pltpu.async_copy(src_hbm.at[pl.ds(off, SUP)],
                                 sbuf.at[nxt], ssem)
                pltpu.async_copy(dst_hbm.at[pl.ds(off, SUP)],
                                 dbuf.at[nxt], dsem)

            @plsc.parallel_loop(0, SUP * (128 // LN), step=1, unroll=4)
            def gbody(t):
                r = lax.shift_right_logical(t, 3)
                s = pl.ds(lax.shift_left(lax.bitwise_and(t, 7), 4), LN)
                sv = sbuf[gmod, r, s]
                dv = dbuf[gmod, r, s]
                for k in range(KPT):
                    vals = plsc.load_gather(hcol[k], [sv])
                    plsc.addupdate_scatter(acol[k], [dv], vals)
            return 0
        lax.fori_loop(0, NSS, body, 0)

        for k in range(KPT):
            pltpu.sync_copy(acol[k], out_hbm.at[wid, k])

    return agg_kernel


def _norms_from_parts(dp):
    deg = jnp.sum(dp, axis=0)
    nrm = lax.rsqrt(jnp.maximum(deg, 1.0))
    return nrm[0:1, :], nrm[1:2, :]


def _mm_scale_body(x_ref, w_ref, dp_ref, o_ref):
    ns, _ = _norms_from_parts(dp_ref[...])
    y = lax.dot_general(w_ref[...], x_ref[...], (((0,), (1,)), ((), ())),
                        preferred_element_type=jnp.float32)
    o_ref[...] = y * ns


def _post_mm_body(a_ref, dp_ref, b_ref, w_ref, o_ref):
    ns, nd = _norms_from_parts(dp_ref[...])
    x = jnp.maximum(a_ref[...] * nd + b_ref[...], 0.0)
    y = lax.dot_general(w_ref[...], x, (((0,), (0,)), ((), ())),
                        preferred_element_type=jnp.float32)
    o_ref[...] = y * ns


def _make_readout_body(NN, NB, H, C):
    def readout_body(a_ref, dp_ref, b_ref, wr_ref, o_ref, acc_ref):
        i = pl.program_id(0)
        _, nd = _norms_from_parts(dp_ref[...])
        x = jnp.maximum(a_ref[...] * nd + b_ref[...], 0.0)
        colid = i * BM + lax.broadcasted_iota(jnp.int32, (H, BM), 1)
        x = jnp.where(colid < NN, x, 0.0)
        s = jnp.sum(x, axis=1, keepdims=True)

        @pl.when(i == 0)
        def _():
            acc_ref[...] = s

        @pl.when(i > 0)
        def _():
            acc_ref[...] = acc_ref[...] + s

        @pl.when(i == NB - 1)
        def _():
            o_ref[...] = lax.dot_general(
                acc_ref[...] / NN, wr_ref[...], (((0,), (0,)), ((), ())),
                preferred_element_type=jnp.float32)
    return readout_body


def kernel(feat, edge_index, W0, b0, W1, b1, Wr):
    NN, D = feat.shape
    E = edge_index.shape[1]
    H = W0.shape[1]
    C = Wr.shape[1]

    NP = -(-(NN + 1) // BM) * BM
    EP = -(-E // (NW * 128 * 16)) * (NW * 128 * 16)
    NR = EP // 128
    RA = NR // NW
    NB = NP // BM

    src = edge_index[0]
    dst = edge_index[1]
    padi = jnp.full((EP - E,), NN, jnp.int32)
    src2d = jnp.concatenate([src, padi]).reshape(NR, 128)
    dst2d = jnp.concatenate([dst, padi]).reshape(NR, 128)
    feat_p = jnp.pad(feat, ((0, NP - NN), (0, 0)))

    dparts = _make_deg_kernel(NP, RA)(src2d, dst2d)

    colT_spec = pl.BlockSpec((H, BM), lambda i: (0, i))
    dp_spec = pl.BlockSpec((NW, 2, BM), lambda i: (0, 0, i))
    w_spec = pl.BlockSpec((D, H), lambda i: (0, 0))
    bT_spec = pl.BlockSpec((H, 1), lambda i: (0, 0))

    hs1 = pl.pallas_call(
        _mm_scale_body,
        grid=(NB,),
        in_specs=[pl.BlockSpec((BM, D), lambda i: (i, 0)), w_spec, dp_spec],
        out_specs=colT_spec,
        out_shape=jax.ShapeDtypeStruct((H, NP), jnp.float32),
    )(feat_p, W0, dparts)

    agg = _make_agg_kernel(NP, NR)
    p1 = agg(hs1.reshape(NW, KPT, NP), src2d, dst2d)

    hs2 = pl.pallas_call(
        _post_mm_body,
        grid=(NB,),
        in_specs=[colT_spec, dp_spec, bT_spec, w_spec],
        out_specs=colT_spec,
        out_shape=jax.ShapeDtypeStruct((H, NP), jnp.float32),
    )(p1.reshape(H, NP), dparts, b0.reshape(H, 1), W1)

    p2 = agg(hs2.reshape(NW, KPT, NP), src2d, dst2d)

    out = pl.pallas_call(
        _make_readout_body(NN, NB, H, C),
        grid=(NB,),
        in_specs=[colT_spec, dp_spec, bT_spec,
                  pl.BlockSpec((H, C), lambda i: (0, 0))],
        out_specs=pl.BlockSpec((1, C), lambda i: (0, 0)),
        out_shape=jax.ShapeDtypeStruct((1, C), jnp.float32),
        scratch_shapes=[pltpu.VMEM((H, 1), jnp.float32)],
    )(p2.reshape(H, NP), dparts, b1.reshape(H, 1), Wr)

    return out


# speedup vs baseline: 1.1418x; 1.1023x over previous
import functools

import jax
import jax.numpy as jnp
from jax import lax
from jax.experimental import pallas as pl
from jax.experimental.pallas import tpu as pltpu
from jax.experimental.pallas import tpu_sc as plsc

NC = 2
NS = 16
NW = NC * NS
LN = 16
BM = 512
SUP = 32
KPT = 4


def _make_deg_kernel(NP, RA):
    mesh = plsc.VectorSubcoreMesh(core_axis_name="c", subcore_axis_name="s",
                                  num_cores=NC, num_subcores=NS)

    @functools.partial(
        pl.kernel, mesh=mesh,
        out_type=jax.ShapeDtypeStruct((NW, 2, NP), jnp.float32),
        scratch_types=[
            pltpu.VMEM((RA, 128), jnp.int32),
            pltpu.VMEM((RA, 128), jnp.int32),
            pltpu.VMEM((NP,), jnp.float32),
            pltpu.VMEM((NP,), jnp.float32),
        ],
        compiler_params=pltpu.CompilerParams(needs_layout_passes=False),
    )
    def deg_kernel(src_hbm, dst_hbm, out_hbm, sidx, didx, dego, degi):
        cid = lax.axis_index("c")
        sid = lax.axis_index("s")
        wid = sid * NC + cid
        pltpu.sync_copy(src_hbm.at[pl.ds(wid * RA, RA)], sidx)
        pltpu.sync_copy(dst_hbm.at[pl.ds(wid * RA, RA)], didx)
        zeros16 = jnp.zeros((LN,), jnp.float32)

        @plsc.parallel_loop(0, NP // LN, step=1, unroll=4)
        def zbody(i):
            dego[pl.ds(i * LN, LN)] = zeros16
            degi[pl.ds(i * LN, LN)] = zeros16

        ones16 = jnp.full((LN,), 1.0, jnp.float32)

        @plsc.parallel_loop(0, RA, step=1, unroll=2)
        def ebody(j):
            for g in range(128 // LN):
                si = sidx[j, pl.ds(g * LN, LN)]
                plsc.addupdate_scatter(dego, [si], ones16)
                di = didx[j, pl.ds(g * LN, LN)]
                plsc.addupdate_scatter(degi, [di], ones16)

        pltpu.sync_copy(dego, out_hbm.at[wid, 0])
        pltpu.sync_copy(degi, out_hbm.at[wid, 1])

    return deg_kernel


def _make_agg_kernel(NP, NR):
    NSS = NR // SUP
    mesh = plsc.VectorSubcoreMesh(core_axis_name="c", subcore_axis_name="s",
                                  num_cores=NC, num_subcores=NS)

    @functools.partial(
        pl.kernel, mesh=mesh,
        out_type=jax.ShapeDtypeStruct((NW, KPT, NP), jnp.float32),
        scratch_types=[pltpu.VMEM((NP,), jnp.int32)] * (KPT // 2)
        + [pltpu.VMEM((NP,), jnp.float32)] * KPT + [
            pltpu.VMEM((2, SUP, 128), jnp.int32),
            pltpu.VMEM((2, SUP, 128), jnp.int32),
            pltpu.SemaphoreType.DMA,
            pltpu.SemaphoreType.DMA,
        ],
        compiler_params=pltpu.CompilerParams(needs_layout_passes=False),
    )
    def agg_kernel(hs_hbm, src_hbm, dst_hbm, out_hbm, *rest):
        hcol = rest[:KPT // 2]
        acol = rest[KPT // 2:KPT // 2 + KPT]
        sbuf, dbuf, ssem, dsem = rest[KPT // 2 + KPT:]
        cid = lax.axis_index("c")
        sid = lax.axis_index("s")
        wid = sid * NC + cid

        for k in range(KPT // 2):
            pltpu.sync_copy(hs_hbm.at[wid, k], hcol[k])

        zeros16 = jnp.zeros((LN,), jnp.float32)

        @plsc.parallel_loop(0, NP // LN, step=1, unroll=4)
        def zbody(i):
            for k in range(KPT):
                acol[k][pl.ds(i * LN, LN)] = zeros16

        pltpu.sync_copy(src_hbm.at[pl.ds(0, SUP)], sbuf.at[0])
        pltpu.sync_copy(dst_hbm.at[pl.ds(0, SUP)], dbuf.at[0])

        def body(g, _):
            gmod = g % 2

            @pl.when(g > 0)
            def _():
                pltpu.make_async_copy(src_hbm.at[pl.ds(0, SUP)],
                                      sbuf.at[0], ssem).wait()
                pltpu.make_async_copy(dst_hbm.at[pl.ds(0, SUP)],
                                      dbuf.at[0], dsem).wait()

            @pl.when(g < NSS - 1)
            def _():
                off = pl.multiple_of((g + 1) * SUP, SUP)
                nxt = (g + 1) % 2
                pltpu.async_copy(src_hbm.at[pl.ds(off, SUP)],
                                 sbuf.at[nxt], ssem)
                pltpu.async_copy(dst_hbm.at[pl.ds(off, SUP)],
                                 dbuf.at[nxt], dsem)

            @plsc.parallel_loop(0, SUP * (128 // LN), step=1, unroll=4)
            def gbody(t):
                r = lax.shift_right_logical(t, 3)
                s = pl.ds(lax.shift_left(lax.bitwise_and(t, 7), 4), LN)
                sv = sbuf[gmod, r, s]
                dv = dbuf[gmod, r, s]
                for k in range(KPT // 2):
                    v = plsc.load_gather(hcol[k], [sv])
                    lo = plsc.bitcast(lax.shift_left(v, 16), jnp.float32)
                    hi = plsc.bitcast(
                        lax.bitwise_and(v, jnp.int32(-65536)), jnp.float32)
                    plsc.addupdate_scatter(acol[2 * k], [dv], lo)
                    plsc.addupdate_scatter(acol[2 * k + 1], [dv], hi)
            return 0
        lax.fori_loop(0, NSS, body, 0)

        for k in range(KPT):
            pltpu.sync_copy(acol[k], out_hbm.at[wid, k])

    return agg_kernel


def _norms_from_parts(dp):
    deg = jnp.sum(dp, axis=0)
    nrm = lax.rsqrt(jnp.maximum(deg, 1.0))
    return nrm[0:1, :], nrm[1:2, :]


def _pack_cols(y, H):
    bits = lax.bitcast_convert_type(y, jnp.int32)
    r16 = lax.shift_right_logical(bits + 0x8000, 16)
    lo = r16[:H // 2]
    hi = r16[H // 2:]
    return lax.bitwise_or(lax.shift_left(hi, 16), lo)


def _mm_scale_body(x_ref, w_ref, dp_ref, o_ref):
    ns, _ = _norms_from_parts(dp_ref[...])
    y = lax.dot_general(w_ref[...], x_ref[...], (((0,), (1,)), ((), ())),
                        preferred_element_type=jnp.float32)
    o_ref[...] = _pack_cols(y * ns, y.shape[0])


def _post_mm_body(a_ref, dp_ref, b_ref, w_ref, o_ref):
    ns, nd = _norms_from_parts(dp_ref[...])
    x = jnp.maximum(a_ref[...] * nd + b_ref[...], 0.0)
    y = lax.dot_general(w_ref[...], x, (((0,), (0,)), ((), ())),
                        preferred_element_type=jnp.float32)
    o_ref[...] = _pack_cols(y * ns, y.shape[0])


def _make_readout_body(NN, NB, H, C):
    def readout_body(a_ref, dp_ref, b_ref, wr_ref, o_ref, acc_ref):
        i = pl.program_id(0)
        _, nd = _norms_from_parts(dp_ref[...])
        x = jnp.maximum(a_ref[...] * nd + b_ref[...], 0.0)
        colid = i * BM + lax.broadcasted_iota(jnp.int32, (H, BM), 1)
        x = jnp.where(colid < NN, x, 0.0)
        s = jnp.sum(x, axis=1, keepdims=True)

        @pl.when(i == 0)
        def _():
            acc_ref[...] = s

        @pl.when(i > 0)
        def _():
            acc_ref[...] = acc_ref[...] + s

        @pl.when(i == NB - 1)
        def _():
            o_ref[...] = lax.dot_general(
                acc_ref[...] / NN, wr_ref[...], (((0,), (0,)), ((), ())),
                preferred_element_type=jnp.float32)
    return readout_body


def kernel(feat, edge_index, W0, b0, W1, b1, Wr):
    NN, D = feat.shape
    E = edge_index.shape[1]
    H = W0.shape[1]
    C = Wr.shape[1]

    NP = -(-(NN + 1) // BM) * BM
    EP = -(-E // (NW * 128 * 16)) * (NW * 128 * 16)
    NR = EP // 128
    RA = NR // NW
    NB = NP // BM

    src = edge_index[0]
    dst = edge_index[1]
    padi = jnp.full((EP - E,), NN, jnp.int32)
    src2d = jnp.concatenate([src, padi]).reshape(NR, 128)
    dst2d = jnp.concatenate([dst, padi]).reshape(NR, 128)
    feat_p = jnp.pad(feat, ((0, NP - NN), (0, 0)))

    dparts = _make_deg_kernel(NP, RA)(src2d, dst2d)

    perm = jnp.asarray(
        [c for w in range(NW)
         for c in (2 * w, 2 * w + H // 2, 2 * w + 1, 2 * w + H // 2 + 1)],
        dtype=jnp.int32)
    W1p = W1[perm, :]
    Wrp = Wr[perm, :]
    b0p = b0[perm]
    b1p = b1[perm]

    colT_spec = pl.BlockSpec((H, BM), lambda i: (0, i))
    pkT_spec = pl.BlockSpec((H // 2, BM), lambda i: (0, i))
    dp_spec = pl.BlockSpec((NW, 2, BM), lambda i: (0, 0, i))
    w_spec = pl.BlockSpec((D, H), lambda i: (0, 0))
    bT_spec = pl.BlockSpec((H, 1), lambda i: (0, 0))

    hs1 = pl.pallas_call(
        _mm_scale_body,
        grid=(NB,),
        in_specs=[pl.BlockSpec((BM, D), lambda i: (i, 0)), w_spec, dp_spec],
        out_specs=pkT_spec,
        out_shape=jax.ShapeDtypeStruct((H // 2, NP), jnp.int32),
    )(feat_p, W0, dparts)

    agg = _make_agg_kernel(NP, NR)
    p1 = agg(hs1.reshape(NW, KPT // 2, NP), src2d, dst2d)

    hs2 = pl.pallas_call(
        _post_mm_body,
        grid=(NB,),
        in_specs=[colT_spec, dp_spec, bT_spec, w_spec],
        out_specs=pkT_spec,
        out_shape=jax.ShapeDtypeStruct((H // 2, NP), jnp.int32),
    )(p1.reshape(H, NP), dparts, b0p.reshape(H, 1), W1p)

    p2 = agg(hs2.reshape(NW, KPT // 2, NP), src2d, dst2d)

    out = pl.pallas_call(
        _make_readout_body(NN, NB, H, C),
        grid=(NB,),
        in_specs=[colT_spec, dp_spec, bT_spec,
                  pl.BlockSpec((H, C), lambda i: (0, 0))],
        out_specs=pl.BlockSpec((1, C), lambda i: (0, 0)),
        out_shape=jax.ShapeDtypeStruct((1, C), jnp.float32),
        scratch_shapes=[pltpu.VMEM((H, 1), jnp.float32)],
    )(p2.reshape(H, NP), dparts, b1p.reshape(H, 1), Wrp)

    return out


# speedup vs baseline: 1.1995x; 1.0505x over previous
import functools

import jax
import jax.numpy as jnp
from jax import lax
from jax.experimental import pallas as pl
from jax.experimental.pallas import tpu as pltpu
from jax.experimental.pallas import tpu_sc as plsc

NC = 2
NS = 16
NW = NC * NS
LN = 16
BM = 512
SUP = 32
KPT = 4


def _make_deg_kernel(NP, RA):
    mesh = plsc.VectorSubcoreMesh(core_axis_name="c", subcore_axis_name="s",
                                  num_cores=NC, num_subcores=NS)

    @functools.partial(
        pl.kernel, mesh=mesh,
        out_type=jax.ShapeDtypeStruct((NW, 2, NP), jnp.float32),
        scratch_types=[
            pltpu.VMEM((RA, 128), jnp.int32),
            pltpu.VMEM((NP,), jnp.float32),
            pltpu.VMEM((NP,), jnp.float32),
        ],
        compiler_params=pltpu.CompilerParams(needs_layout_passes=False),
    )
    def deg_kernel(edg_hbm, out_hbm, eidx, dego, degi):
        cid = lax.axis_index("c")
        sid = lax.axis_index("s")
        wid = sid * NC + cid
        pltpu.sync_copy(edg_hbm.at[pl.ds(wid * RA, RA)], eidx)
        zeros16 = jnp.zeros((LN,), jnp.float32)

        @plsc.parallel_loop(0, NP // LN, step=1, unroll=4)
        def zbody(i):
            dego[pl.ds(i * LN, LN)] = zeros16
            degi[pl.ds(i * LN, LN)] = zeros16

        ones16 = jnp.full((LN,), 1.0, jnp.float32)

        @plsc.parallel_loop(0, RA, step=1, unroll=2)
        def ebody(j):
            for g in range(128 // LN):
                ev = eidx[j, pl.ds(g * LN, LN)]
                si = lax.bitwise_and(ev, jnp.int32(0xFFFF))
                di = lax.shift_right_logical(ev, 16)
                plsc.addupdate_scatter(dego, [si], ones16)
                plsc.addupdate_scatter(degi, [di], ones16)

        pltpu.sync_copy(dego, out_hbm.at[wid, 0])
        pltpu.sync_copy(degi, out_hbm.at[wid, 1])

    return deg_kernel


def _make_agg_kernel(NP, NR):
    NSS = NR // SUP
    mesh = plsc.VectorSubcoreMesh(core_axis_name="c", subcore_axis_name="s",
                                  num_cores=NC, num_subcores=NS)

    @functools.partial(
        pl.kernel, mesh=mesh,
        out_type=jax.ShapeDtypeStruct((NW, KPT, NP), jnp.float32),
        scratch_types=[pltpu.VMEM((NP,), jnp.int32)] * (KPT // 2)
        + [pltpu.VMEM((NP,), jnp.float32)] * KPT + [
            pltpu.VMEM((2, SUP, 128), jnp.int32),
            pltpu.SemaphoreType.DMA,
        ],
        compiler_params=pltpu.CompilerParams(needs_layout_passes=False),
    )
    def agg_kernel(hs_hbm, edg_hbm, out_hbm, *rest):
        hcol = rest[:KPT // 2]
        acol = rest[KPT // 2:KPT // 2 + KPT]
        ebuf, esem = rest[KPT // 2 + KPT:]
        cid = lax.axis_index("c")
        sid = lax.axis_index("s")
        wid = sid * NC + cid

        for k in range(KPT // 2):
            pltpu.sync_copy(hs_hbm.at[wid, k], hcol[k])

        zeros16 = jnp.zeros((LN,), jnp.float32)

        @plsc.parallel_loop(0, NP // LN, step=1, unroll=4)
        def zbody(i):
            for k in range(KPT):
                acol[k][pl.ds(i * LN, LN)] = zeros16

        pltpu.sync_copy(edg_hbm.at[pl.ds(0, SUP)], ebuf.at[0])

        def body(g, _):
            gmod = g % 2

            @pl.when(g > 0)
            def _():
                pltpu.make_async_copy(edg_hbm.at[pl.ds(0, SUP)],
                                      ebuf.at[0], esem).wait()

            @pl.when(g < NSS - 1)
            def _():
                off = pl.multiple_of((g + 1) * SUP, SUP)
                pltpu.async_copy(edg_hbm.at[pl.ds(off, SUP)],
                                 ebuf.at[(g + 1) % 2], esem)

            @plsc.parallel_loop(0, SUP * (128 // LN), step=1, unroll=4)
            def gbody(t):
                r = lax.shift_right_logical(t, 3)
                s = pl.ds(lax.shift_left(lax.bitwise_and(t, 7), 4), LN)
                ev = ebuf[gmod, r, s]
                sv = lax.bitwise_and(ev, jnp.int32(0xFFFF))
                dv = lax.shift_right_logical(ev, 16)
                for k in range(KPT // 2):
                    v = plsc.load_gather(hcol[k], [sv])
                    lo = plsc.bitcast(lax.shift_left(v, 16), jnp.float32)
                    hi = plsc.bitcast(
                        lax.bitwise_and(v, jnp.int32(-65536)), jnp.float32)
                    plsc.addupdate_scatter(acol[2 * k], [dv], lo)
                    plsc.addupdate_scatter(acol[2 * k + 1], [dv], hi)
            return 0
        lax.fori_loop(0, NSS, body, 0)

        for k in range(KPT):
            pltpu.sync_copy(acol[k], out_hbm.at[wid, k])

    return agg_kernel


def _norms_from_parts(dp):
    deg = jnp.sum(dp, axis=0)
    nrm = lax.rsqrt(jnp.maximum(deg, 1.0))
    return nrm[0:1, :], nrm[1:2, :]


def _pack_cols(y, H):
    bits = lax.bitcast_convert_type(y, jnp.int32)
    r16 = lax.shift_right_logical(bits + 0x8000, 16)
    lo = r16[:H // 2]
    hi = r16[H // 2:]
    return lax.bitwise_or(lax.shift_left(hi, 16), lo)


def _mm_scale_body(x_ref, w_ref, dp_ref, o_ref):
    ns, _ = _norms_from_parts(dp_ref[...])
    y = lax.dot_general(w_ref[...], x_ref[...], (((0,), (1,)), ((), ())),
                        preferred_element_type=jnp.float32)
    o_ref[...] = _pack_cols(y * ns, y.shape[0])


def _post_mm_body(a_ref, dp_ref, b_ref, w_ref, o_ref):
    ns, nd = _norms_from_parts(dp_ref[...])
    x = jnp.maximum(a_ref[...] * nd + b_ref[...], 0.0)
    y = lax.dot_general(w_ref[...], x, (((0,), (0,)), ((), ())),
                        preferred_element_type=jnp.float32)
    o_ref[...] = _pack_cols(y * ns, y.shape[0])


def _make_readout_body(NN, NB, H, C):
    def readout_body(a_ref, dp_ref, b_ref, wr_ref, o_ref, acc_ref):
        i = pl.program_id(0)
        _, nd = _norms_from_parts(dp_ref[...])
        x = jnp.maximum(a_ref[...] * nd + b_ref[...], 0.0)
        colid = i * BM + lax.broadcasted_iota(jnp.int32, (H, BM), 1)
        x = jnp.where(colid < NN, x, 0.0)
        s = jnp.sum(x, axis=1, keepdims=True)

        @pl.when(i == 0)
        def _():
            acc_ref[...] = s

        @pl.when(i > 0)
        def _():
            acc_ref[...] = acc_ref[...] + s

        @pl.when(i == NB - 1)
        def _():
            o_ref[...] = lax.dot_general(
                acc_ref[...] / NN, wr_ref[...], (((0,), (0,)), ((), ())),
                preferred_element_type=jnp.float32)
    return readout_body


def kernel(feat, edge_index, W0, b0, W1, b1, Wr):
    NN, D = feat.shape
    E = edge_index.shape[1]
    H = W0.shape[1]
    C = Wr.shape[1]

    NP = -(-(NN + 1) // BM) * BM
    EP = -(-E // (NW * 128 * 16)) * (NW * 128 * 16)
    NR = EP // 128
    RA = NR // NW
    NB = NP // BM

    src = edge_index[0]
    dst = edge_index[1]
    padi = jnp.full((EP - E,), NN, jnp.int32)
    srcp = jnp.concatenate([src, padi])
    dstp = jnp.concatenate([dst, padi])
    edg2d = jnp.bitwise_or(srcp, dstp << 16).reshape(NR, 128)
    feat_p = jnp.pad(feat, ((0, NP - NN), (0, 0)))

    dparts = _make_deg_kernel(NP, RA)(edg2d)

    perm = jnp.asarray(
        [c for w in range(NW)
         for c in (2 * w, 2 * w + H // 2, 2 * w + 1, 2 * w + H // 2 + 1)],
        dtype=jnp.int32)
    W1p = W1[perm, :]
    Wrp = Wr[perm, :]
    b0p = b0[perm]
    b1p = b1[perm]

    colT_spec = pl.BlockSpec((H, BM), lambda i: (0, i))
    pkT_spec = pl.BlockSpec((H // 2, BM), lambda i: (0, i))
    dp_spec = pl.BlockSpec((NW, 2, BM), lambda i: (0, 0, i))
    w_spec = pl.BlockSpec((D, H), lambda i: (0, 0))
    bT_spec = pl.BlockSpec((H, 1), lambda i: (0, 0))

    hs1 = pl.pallas_call(
        _mm_scale_body,
        grid=(NB,),
        in_specs=[pl.BlockSpec((BM, D), lambda i: (i, 0)), w_spec, dp_spec],
        out_specs=pkT_spec,
        out_shape=jax.ShapeDtypeStruct((H // 2, NP), jnp.int32),
    )(feat_p, W0, dparts)

    agg = _make_agg_kernel(NP, NR)
    p1 = agg(hs1.reshape(NW, KPT // 2, NP), edg2d)

    hs2 = pl.pallas_call(
        _post_mm_body,
        grid=(NB,),
        in_specs=[colT_spec, dp_spec, bT_spec, w_spec],
        out_specs=pkT_spec,
        out_shape=jax.ShapeDtypeStruct((H // 2, NP), jnp.int32),
    )(p1.reshape(H, NP), dparts, b0p.reshape(H, 1), W1p)

    p2 = agg(hs2.reshape(NW, KPT // 2, NP), edg2d)

    out = pl.pallas_call(
        _make_readout_body(NN, NB, H, C),
        grid=(NB,),
        in_specs=[colT_spec, dp_spec, bT_spec,
                  pl.BlockSpec((H, C), lambda i: (0, 0))],
        out_specs=pl.BlockSpec((1, C), lambda i: (0, 0)),
        out_shape=jax.ShapeDtypeStruct((1, C), jnp.float32),
        scratch_shapes=[pltpu.VMEM((H, 1), jnp.float32)],
    )(p2.reshape(H, NP), dparts, b1p.reshape(H, 1), Wrp)

    return out
